# baseline (device time: 14541 ns/iter reference)
import jax
import jax.numpy as jnp
from jax import lax
from jax.experimental import pallas as pl
from jax.experimental.pallas import tpu as pltpu

T, D, V = 512, 1024, 8192
NDEV = 16
NSLICE = 8
SW = V // NSLICE
HW = SW // 2


def kernel(x, W, labels):
    labels2d = labels.reshape(T, 1)

    def body(x_ref, w_ref, lab_ref, out_ref,
             x_vmem, w_vmem, comm_send, comm_recv,
             x_sem, w_sems, send_sems, recv_sems):
        my_x = lax.axis_index("x")
        my_y = lax.axis_index("y")
        my_z = lax.axis_index("z")
        mine = my_x * 8 + my_y * 4 + my_z
        k = my_y * 4 + my_z

        xcp = pltpu.make_async_copy(x_ref, x_vmem, x_sem)
        xcp.start()
        wcps = []
        for h in range(2):
            cp = pltpu.make_async_copy(
                w_ref.at[:, pl.ds(k * SW + h * HW, HW)],
                w_vmem.at[:, pl.ds(h * HW, HW)],
                w_sems.at[h])
            cp.start()
            wcps.append(cp)

        def peer(d):
            tgt = lax.rem(mine + d, NDEV)
            return (tgt // 8, (tgt // 4) % 2, tgt % 4)

        barrier = pltpu.get_barrier_semaphore()
        for d in range(1, NDEV):
            pl.semaphore_signal(barrier, inc=1, device_id=peer(d),
                                device_id_type=pl.DeviceIdType.MESH)

        xcp.wait()
        xb = x_vmem[...].astype(jnp.bfloat16)
        cols = lax.broadcasted_iota(jnp.int32, (T, HW), 1)
        s = jnp.zeros((T, 1), jnp.float32)
        p = jnp.zeros((T, 1), jnp.float32)
        for h in range(2):
            wcps[h].wait()
            wb = w_vmem[:, h * HW:(h + 1) * HW].astype(jnp.bfloat16)
            lg = jnp.dot(xb, wb, preferred_element_type=jnp.float32)
            s = s + jnp.sum(jnp.exp(lg), axis=1, keepdims=True)
            local_id = lab_ref[...] - (my_x * V + k * SW + h * HW)
            p = p + jnp.sum(jnp.where(cols == local_id, lg, 0.0),
                            axis=1, keepdims=True)
        comm_send[...] = jnp.concatenate([s, p], axis=1).T

        pl.semaphore_wait(barrier, NDEV - 1)

        rdmas = []
        for d in range(1, NDEV):
            rdma = pltpu.make_async_remote_copy(
                src_ref=comm_send,
                dst_ref=comm_recv.at[d - 1],
                send_sem=send_sems.at[d - 1],
                recv_sem=recv_sems.at[d - 1],
                device_id=peer(d),
                device_id_type=pl.DeviceIdType.MESH)
            rdma.start()
            rdmas.append(rdma)
        for rdma in rdmas:
            rdma.wait_send()
        for rdma in rdmas:
            rdma.wait_recv()

        tot = comm_send[...] + jnp.sum(comm_recv[...], axis=0)
        out_ref[...] = jnp.log(tot[0:1, :]) - tot[1:2, :]

    out = pl.pallas_call(
        body,
        out_shape=jax.ShapeDtypeStruct((1, T), jnp.float32),
        in_specs=[
            pl.BlockSpec(memory_space=pl.ANY),
            pl.BlockSpec(memory_space=pl.ANY),
            pl.BlockSpec(memory_space=pltpu.VMEM),
        ],
        out_specs=pl.BlockSpec(memory_space=pltpu.VMEM),
        scratch_shapes=[
            pltpu.VMEM((T, D), jnp.float32),
            pltpu.VMEM((D, SW), jnp.float32),
            pltpu.VMEM((2, T), jnp.float32),
            pltpu.VMEM((NDEV - 1, 2, T), jnp.float32),
            pltpu.SemaphoreType.DMA,
            pltpu.SemaphoreType.DMA((2,)),
            pltpu.SemaphoreType.DMA((NDEV - 1,)),
            pltpu.SemaphoreType.DMA((NDEV - 1,)),
        ],
        compiler_params=pltpu.CompilerParams(collective_id=0),
    )(x, W, labels2d)
    return out.reshape(T)
